# Initial kernel scaffold; baseline (speedup 1.0000x reference)
import jax
import jax.numpy as jnp
from jax.experimental import pallas as pl


def kernel(roi_bboxes, gt_boxes, gt_labels):
    raise NotImplementedError("write your pallas kernel here")


# speedup vs baseline: 1.4291x; 1.4291x over previous
import jax
import jax.numpy as jnp
from jax import lax
from jax.experimental import pallas as pl
from jax.experimental.pallas import tpu as pltpu

_NUM_LABELS = 81
_POS_K = 64
_NEG_K = 192
_TILE = 2000


def _iou_gather_kernel(roi_ref, gtt_ref, gtl_ref, rp_ref, rn_ref,
                       mp_ref, mn_ref, pack_ref):
    r = roi_ref[0, 0]
    by1 = r[:, 0:1]
    bx1 = r[:, 1:2]
    by2 = r[:, 2:3]
    bx2 = r[:, 3:4]
    g = gtt_ref[0]
    gy1 = g[0:1, :]
    gx1 = g[1:2, :]
    gy2 = g[2:3, :]
    gx2 = g[3:4, :]
    x_top = jnp.maximum(bx1, gx1)
    y_top = jnp.maximum(by1, gy1)
    x_bot = jnp.minimum(bx2, gx2)
    y_bot = jnp.minimum(by2, gy2)
    inter = jnp.maximum(x_bot - x_top, 0.0) * jnp.maximum(y_bot - y_top, 0.0)
    barea = (by2 - by1) * (bx2 - bx1)
    iou = inter / (barea - inter)
    m = iou.shape[1]
    mx = jnp.max(iou, axis=1, keepdims=True)
    eqm = iou == mx
    i0 = lax.broadcasted_iota(jnp.int32, iou.shape, 1)
    am = jnp.min(jnp.where(eqm, i0, m), axis=1, keepdims=True)
    onehot = (i0 == am).astype(jnp.float32)
    lab = jnp.sum(onehot * gtl_ref[0], axis=1, keepdims=True)
    gby1 = jnp.sum(onehot * gy1, axis=1, keepdims=True)
    gbx1 = jnp.sum(onehot * gx1, axis=1, keepdims=True)
    gby2 = jnp.sum(onehot * gy2, axis=1, keepdims=True)
    gbx2 = jnp.sum(onehot * gx2, axis=1, keepdims=True)
    pos_c = mx > 0.5
    neg_c = jnp.logical_and(mx < 0.5, mx > 0.1)
    mp_ref[0, 0] = jnp.where(pos_c, rp_ref[0, 0], 0)
    mn_ref[0, 0] = jnp.where(neg_c, rn_ref[0, 0], 0)
    pack_ref[0, 0] = jnp.concatenate([lab, gby1, gbx1, gby2, gbx2], axis=1)


def _select_thresholds(mfull, gidx, k_sel, hi0, n):
    def cnt_gt(t):
        return jnp.sum((mfull > t).astype(jnp.int32))

    def body_t(_, c):
        lo, hi = c
        mid = (lo + hi) // 2
        pred = cnt_gt(mid) < k_sel
        return (jnp.where(pred, lo, mid + 1), jnp.where(pred, mid, hi))

    lo, _ = lax.fori_loop(0, 11, body_t, (jnp.int32(0), jnp.int32(hi0)))
    t_val = lo
    need = k_sel - cnt_gt(t_val)
    eq = mfull == t_val

    def cnt_eq_lt(i):
        return jnp.sum(jnp.where(jnp.logical_and(eq, gidx < i), 1, 0))

    def body_i(_, c):
        lo2, hi2 = c
        mid = (lo2 + hi2) // 2
        pred = cnt_eq_lt(mid) >= need
        return (jnp.where(pred, lo2, mid + 1), jnp.where(pred, mid, hi2))

    lo2, _ = lax.fori_loop(0, 15, body_i, (jnp.int32(0), jnp.int32(n)))
    return t_val, lo2


def _threshold_kernel(mpf_ref, mnf_ref, thr_ref):
    mpf = mpf_ref[0]
    mnf = mnf_ref[0]
    nt, _, tile = mpf.shape
    n = nt * tile
    gidx = (lax.broadcasted_iota(jnp.int32, mpf.shape, 0) * tile
            + lax.broadcasted_iota(jnp.int32, mpf.shape, 2))
    t_p, i_p = _select_thresholds(mpf, gidx, _POS_K, _POS_K * 10 - 1, n)
    t_n, i_n = _select_thresholds(mnf, gidx, _NEG_K, _NEG_K * 10 - 1, n)
    i32 = jnp.int32
    thr_ref[0] = jnp.concatenate(
        [jnp.full((1, 1), t_p, i32), jnp.full((1, 1), i_p, i32),
         jnp.full((1, 1), t_n, i32), jnp.full((1, 1), i_n, i32)], axis=1)


def _assign_kernel(roi_ref, mp_ref, mn_ref, pack_ref, thr_ref,
                   lout_ref, dout_ref):
    k = pl.program_id(1)
    thr = thr_ref[0]
    t_p = thr[0:1, 0:1]
    i_p = thr[0:1, 1:2]
    t_n = thr[0:1, 2:3]
    i_n = thr[0:1, 3:4]
    mp = mp_ref[0, 0]
    mn = mn_ref[0, 0]
    tile = mp.shape[0]
    gidx = k * tile + lax.broadcasted_iota(jnp.int32, (tile, 1), 0)
    sel_p = (mp > t_p) & (mp > 0) | ((mp == t_p) & (gidx < i_p) & (mp > 0))
    sel_n = (mn > t_n) & (mn > 0) | ((mn == t_n) & (gidx < i_n) & (mn > 0))
    pack = pack_ref[0, 0]
    lbl = jnp.where(sel_p, pack[:, 0:1], jnp.where(sel_n, 0.0, -1.0))
    i81 = lax.broadcasted_iota(jnp.int32, (tile, _NUM_LABELS), 1)
    lout_ref[0] = (i81.astype(jnp.float32) == lbl).astype(jnp.float32)

    r = roi_ref[0, 0]
    w = r[:, 3:4] - r[:, 1:2]
    h = r[:, 2:3] - r[:, 0:1]
    cx = r[:, 1:2] + 0.5 * w
    cy = r[:, 0:1] + 0.5 * h
    egt = jnp.where(sel_p, pack[:, 1:5], 0.0)
    gw = egt[:, 3:4] - egt[:, 1:2]
    gh = egt[:, 2:3] - egt[:, 0:1]
    gcx = egt[:, 1:2] + 0.5 * gw
    gcy = egt[:, 0:1] + 0.5 * gh
    ws = jnp.where(w == 0, 0.001, w)
    hs = jnp.where(h == 0, 0.001, h)
    gws = jnp.where(gw == 0, 1.0, gw)
    ghs = jnp.where(gh == 0, 1.0, gh)
    zero = jnp.zeros_like(gw)
    d_x = jnp.where(gw == 0, zero, (gcx - cx) / ws)
    d_y = jnp.where(gh == 0, zero, (gcy - cy) / hs)
    d_w = jnp.where(gw == 0, zero, jnp.log(gws / ws))
    d_h = jnp.where(gh == 0, zero, jnp.log(ghs / hs))
    i324 = lax.broadcasted_iota(jnp.int32, (tile, _NUM_LABELS * 4), 1)
    j4 = i324 % 4
    dfull = jnp.where(j4 == 0, d_y,
                      jnp.where(j4 == 1, d_x,
                                jnp.where(j4 == 2, d_h, d_w)))
    cls = (i324 // 4).astype(jnp.float32)
    dout_ref[0] = jnp.where(cls == lbl, dfull, 0.0)


def kernel(roi_bboxes, gt_boxes, gt_labels):
    b, n, _ = roi_bboxes.shape
    m = gt_boxes.shape[1]
    tile = _TILE
    nt = n // tile
    f32 = jnp.float32
    i32 = jnp.int32

    kp, kn = jax.random.split(jax.random.key(42))
    r_pos = jax.random.randint(kp, (b, n), 1, _POS_K * 10, dtype=i32)
    r_neg = jax.random.randint(kn, (b, n), 1, _NEG_K * 10, dtype=i32)
    r_pos = r_pos.reshape(b, nt, tile, 1)
    r_neg = r_neg.reshape(b, nt, tile, 1)

    roi_col = roi_bboxes.reshape(b, nt, tile, 4)
    gt_t = gt_boxes.transpose(0, 2, 1)
    gtl_f = gt_labels.astype(f32).reshape(b, 1, m)

    mp, mn, pack = pl.pallas_call(
        _iou_gather_kernel,
        grid=(b, nt),
        in_specs=[
            pl.BlockSpec((1, 1, tile, 4), lambda bi, ki: (bi, ki, 0, 0)),
            pl.BlockSpec((1, 4, m), lambda bi, ki: (bi, 0, 0)),
            pl.BlockSpec((1, 1, m), lambda bi, ki: (bi, 0, 0)),
            pl.BlockSpec((1, 1, tile, 1), lambda bi, ki: (bi, ki, 0, 0)),
            pl.BlockSpec((1, 1, tile, 1), lambda bi, ki: (bi, ki, 0, 0)),
        ],
        out_specs=[
            pl.BlockSpec((1, 1, tile, 1), lambda bi, ki: (bi, ki, 0, 0)),
            pl.BlockSpec((1, 1, tile, 1), lambda bi, ki: (bi, ki, 0, 0)),
            pl.BlockSpec((1, 1, tile, 5), lambda bi, ki: (bi, ki, 0, 0)),
        ],
        out_shape=[
            jax.ShapeDtypeStruct((b, nt, tile, 1), i32),
            jax.ShapeDtypeStruct((b, nt, tile, 1), i32),
            jax.ShapeDtypeStruct((b, nt, tile, 5), f32),
        ],
        compiler_params=pltpu.CompilerParams(
            dimension_semantics=("parallel", "parallel")),
    )(roi_col, gt_t, gtl_f, r_pos, r_neg)

    mp_row = mp.reshape(b, nt, 1, tile)
    mn_row = mn.reshape(b, nt, 1, tile)

    thr = pl.pallas_call(
        _threshold_kernel,
        grid=(b,),
        in_specs=[
            pl.BlockSpec((1, nt, 1, tile), lambda bi: (bi, 0, 0, 0)),
            pl.BlockSpec((1, nt, 1, tile), lambda bi: (bi, 0, 0, 0)),
        ],
        out_specs=pl.BlockSpec((1, 1, 4), lambda bi: (bi, 0, 0)),
        out_shape=jax.ShapeDtypeStruct((b, 1, 4), i32),
        compiler_params=pltpu.CompilerParams(
            dimension_semantics=("arbitrary",)),
    )(mp_row, mn_row)

    labels_out, deltas_flat = pl.pallas_call(
        _assign_kernel,
        grid=(b, nt),
        in_specs=[
            pl.BlockSpec((1, 1, tile, 4), lambda bi, ki: (bi, ki, 0, 0)),
            pl.BlockSpec((1, 1, tile, 1), lambda bi, ki: (bi, ki, 0, 0)),
            pl.BlockSpec((1, 1, tile, 1), lambda bi, ki: (bi, ki, 0, 0)),
            pl.BlockSpec((1, 1, tile, 5), lambda bi, ki: (bi, ki, 0, 0)),
            pl.BlockSpec((1, 1, 4), lambda bi, ki: (bi, 0, 0)),
        ],
        out_specs=[
            pl.BlockSpec((1, tile, _NUM_LABELS), lambda bi, ki: (bi, ki, 0)),
            pl.BlockSpec((1, tile, _NUM_LABELS * 4),
                         lambda bi, ki: (bi, ki, 0)),
        ],
        out_shape=[
            jax.ShapeDtypeStruct((b, n, _NUM_LABELS), f32),
            jax.ShapeDtypeStruct((b, n, _NUM_LABELS * 4), f32),
        ],
        compiler_params=pltpu.CompilerParams(
            dimension_semantics=("parallel", "parallel")),
    )(roi_col, mp, mn, pack, thr)

    deltas = deltas_flat.reshape(b, n, _NUM_LABELS, 4)
    return deltas, labels_out


# speedup vs baseline: 7.9019x; 5.5293x over previous
import jax
import jax.numpy as jnp
from jax import lax
from jax.experimental import pallas as pl
from jax.experimental.pallas import tpu as pltpu

_NUM_LABELS = 81
_POS_K = 64
_NEG_K = 192
_TILE = 1024


def _iou_gather_kernel(roi_ref, gt_ref, gtl_ref, rp_ref, rn_ref,
                       mp_ref, mn_ref, pack_ref):
    r = roi_ref[0, 0]
    by1 = r[0:1, :]
    bx1 = r[1:2, :]
    by2 = r[2:3, :]
    bx2 = r[3:4, :]
    g = gt_ref[0]
    gy1 = g[:, 0:1]
    gx1 = g[:, 1:2]
    gy2 = g[:, 2:3]
    gx2 = g[:, 3:4]
    x_top = jnp.maximum(bx1, gx1)
    y_top = jnp.maximum(by1, gy1)
    x_bot = jnp.minimum(bx2, gx2)
    y_bot = jnp.minimum(by2, gy2)
    inter = jnp.maximum(x_bot - x_top, 0.0) * jnp.maximum(y_bot - y_top, 0.0)
    barea = (by2 - by1) * (bx2 - bx1)
    iou = inter / (barea - inter)
    m = iou.shape[0]
    mx = jnp.max(iou, axis=0, keepdims=True)
    eqm = iou == mx
    i0 = lax.broadcasted_iota(jnp.int32, iou.shape, 0)
    am = jnp.min(jnp.where(eqm, i0, m), axis=0, keepdims=True)
    onehot = (i0 == am).astype(jnp.float32)
    lab = jnp.sum(onehot * gtl_ref[0], axis=0, keepdims=True)
    gby1 = jnp.sum(onehot * gy1, axis=0, keepdims=True)
    gbx1 = jnp.sum(onehot * gx1, axis=0, keepdims=True)
    gby2 = jnp.sum(onehot * gy2, axis=0, keepdims=True)
    gbx2 = jnp.sum(onehot * gx2, axis=0, keepdims=True)
    pos_c = mx > 0.5
    neg_c = jnp.logical_and(mx < 0.5, mx > 0.1)
    mp_ref[0, 0] = jnp.where(pos_c, rp_ref[0, 0], 0)
    mn_ref[0, 0] = jnp.where(neg_c, rn_ref[0, 0], 0)
    pack_ref[0, 0] = jnp.concatenate([lab, gby1, gbx1, gby2, gbx2], axis=0)


def _select_thresholds(mfull, gidx, k_sel, hi0, n):
    def cnt_gt(t):
        return jnp.sum((mfull > t).astype(jnp.int32))

    def body_t(_, c):
        lo, hi = c
        mid = (lo + hi) // 2
        pred = cnt_gt(mid) < k_sel
        return (jnp.where(pred, lo, mid + 1), jnp.where(pred, mid, hi))

    lo, _ = lax.fori_loop(0, 11, body_t, (jnp.int32(0), jnp.int32(hi0)))
    t_val = lo
    need = k_sel - cnt_gt(t_val)
    eq = mfull == t_val

    def cnt_eq_lt(i):
        return jnp.sum(jnp.where(jnp.logical_and(eq, gidx < i), 1, 0))

    def body_i(_, c):
        lo2, hi2 = c
        mid = (lo2 + hi2) // 2
        pred = cnt_eq_lt(mid) >= need
        return (jnp.where(pred, lo2, mid + 1), jnp.where(pred, mid, hi2))

    lo2, _ = lax.fori_loop(0, 15, body_i, (jnp.int32(0), jnp.int32(n)))
    return t_val, lo2


def _threshold_kernel(mpf_ref, mnf_ref, thr_ref):
    mpf = mpf_ref[0]
    mnf = mnf_ref[0]
    nt, _, tile = mpf.shape
    n = nt * tile
    gidx = (lax.broadcasted_iota(jnp.int32, mpf.shape, 0) * tile
            + lax.broadcasted_iota(jnp.int32, mpf.shape, 2))
    t_p, i_p = _select_thresholds(mpf, gidx, _POS_K, _POS_K * 10 - 1, n)
    t_n, i_n = _select_thresholds(mnf, gidx, _NEG_K, _NEG_K * 10 - 1, n)
    i32 = jnp.int32
    thr_ref[0] = jnp.concatenate(
        [jnp.full((1, 1), t_p, i32), jnp.full((1, 1), i_p, i32),
         jnp.full((1, 1), t_n, i32), jnp.full((1, 1), i_n, i32)], axis=1)


def _assign_kernel(roi_ref, mp_ref, mn_ref, pack_ref, thr_ref,
                   lout_ref, dout_ref):
    k = pl.program_id(0)
    thr = thr_ref[...][:, 0, :]
    t_p = thr[:, 0:1]
    i_p = thr[:, 1:2]
    t_n = thr[:, 2:3]
    i_n = thr[:, 3:4]
    mp = mp_ref[...][:, 0, 0, :]
    mn = mn_ref[...][:, 0, 0, :]
    b, tile = mp.shape
    gidx = k * tile + lax.broadcasted_iota(jnp.int32, (b, tile), 1)
    sel_p = (mp > t_p) & (mp > 0) | ((mp == t_p) & (gidx < i_p) & (mp > 0))
    sel_n = (mn > t_n) & (mn > 0) | ((mn == t_n) & (gidx < i_n) & (mn > 0))
    pack = pack_ref[...][:, 0]
    lbl = jnp.where(sel_p, pack[:, 0, :], jnp.where(sel_n, 0.0, -1.0))
    i81 = lax.broadcasted_iota(jnp.int32, (_NUM_LABELS, b, tile), 0)
    lout_ref[...] = (i81.astype(jnp.float32) == lbl[None]).astype(jnp.float32)

    r = roi_ref[...][:, 0]
    w = r[:, 3, :] - r[:, 1, :]
    h = r[:, 2, :] - r[:, 0, :]
    cx = r[:, 1, :] + 0.5 * w
    cy = r[:, 0, :] + 0.5 * h
    gy1 = jnp.where(sel_p, pack[:, 1, :], 0.0)
    gx1 = jnp.where(sel_p, pack[:, 2, :], 0.0)
    gy2 = jnp.where(sel_p, pack[:, 3, :], 0.0)
    gx2 = jnp.where(sel_p, pack[:, 4, :], 0.0)
    gw = gx2 - gx1
    gh = gy2 - gy1
    gcx = gx1 + 0.5 * gw
    gcy = gy1 + 0.5 * gh
    ws = jnp.where(w == 0, 0.001, w)
    hs = jnp.where(h == 0, 0.001, h)
    gws = jnp.where(gw == 0, 1.0, gw)
    ghs = jnp.where(gh == 0, 1.0, gh)
    zero = jnp.zeros_like(gw)
    d_x = jnp.where(gw == 0, zero, (gcx - cx) / ws)
    d_y = jnp.where(gh == 0, zero, (gcy - cy) / hs)
    d_w = jnp.where(gw == 0, zero, jnp.log(gws / ws))
    d_h = jnp.where(gh == 0, zero, jnp.log(ghs / hs))
    jj = lax.broadcasted_iota(jnp.int32, (b, 1, 4, tile), 2)
    dval = jnp.where(jj == 0, d_y[:, None, None, :],
                     jnp.where(jj == 1, d_x[:, None, None, :],
                               jnp.where(jj == 2, d_h[:, None, None, :],
                                         d_w[:, None, None, :])))
    cc = lax.broadcasted_iota(jnp.int32, (b, _NUM_LABELS, 1, tile), 1)
    sel_cls = cc == lbl[:, None, None, :].astype(jnp.int32)
    dout_ref[...] = jnp.where(sel_cls, dval, 0.0)


def kernel(roi_bboxes, gt_boxes, gt_labels):
    b, n, _ = roi_bboxes.shape
    m = gt_boxes.shape[1]
    tile = _TILE
    np_ = ((n + tile - 1) // tile) * tile
    nt = np_ // tile
    f32 = jnp.float32
    i32 = jnp.int32

    kp, kn = jax.random.split(jax.random.key(42))
    r_pos = jax.random.randint(kp, (b, n), 1, _POS_K * 10, dtype=i32)
    r_neg = jax.random.randint(kn, (b, n), 1, _NEG_K * 10, dtype=i32)
    r_pos = jnp.pad(r_pos, ((0, 0), (0, np_ - n))).reshape(b, nt, 1, tile)
    r_neg = jnp.pad(r_neg, ((0, 0), (0, np_ - n))).reshape(b, nt, 1, tile)

    roi_row = (jnp.pad(roi_bboxes, ((0, 0), (0, np_ - n), (0, 0)))
               .reshape(b, nt, tile, 4).transpose(0, 1, 3, 2))
    gtl_f = gt_labels.astype(f32).reshape(b, m, 1)

    mp, mn, pack = pl.pallas_call(
        _iou_gather_kernel,
        grid=(b, nt),
        in_specs=[
            pl.BlockSpec((1, 1, 4, tile), lambda bi, ki: (bi, ki, 0, 0)),
            pl.BlockSpec((1, m, 4), lambda bi, ki: (bi, 0, 0)),
            pl.BlockSpec((1, m, 1), lambda bi, ki: (bi, 0, 0)),
            pl.BlockSpec((1, 1, 1, tile), lambda bi, ki: (bi, ki, 0, 0)),
            pl.BlockSpec((1, 1, 1, tile), lambda bi, ki: (bi, ki, 0, 0)),
        ],
        out_specs=[
            pl.BlockSpec((1, 1, 1, tile), lambda bi, ki: (bi, ki, 0, 0)),
            pl.BlockSpec((1, 1, 1, tile), lambda bi, ki: (bi, ki, 0, 0)),
            pl.BlockSpec((1, 1, 5, tile), lambda bi, ki: (bi, ki, 0, 0)),
        ],
        out_shape=[
            jax.ShapeDtypeStruct((b, nt, 1, tile), i32),
            jax.ShapeDtypeStruct((b, nt, 1, tile), i32),
            jax.ShapeDtypeStruct((b, nt, 5, tile), f32),
        ],
        compiler_params=pltpu.CompilerParams(
            dimension_semantics=("parallel", "parallel")),
    )(roi_row, gt_boxes, gtl_f, r_pos, r_neg)

    thr = pl.pallas_call(
        _threshold_kernel,
        grid=(b,),
        in_specs=[
            pl.BlockSpec((1, nt, 1, tile), lambda bi: (bi, 0, 0, 0)),
            pl.BlockSpec((1, nt, 1, tile), lambda bi: (bi, 0, 0, 0)),
        ],
        out_specs=pl.BlockSpec((1, 1, 4), lambda bi: (bi, 0, 0)),
        out_shape=jax.ShapeDtypeStruct((b, 1, 4), i32),
        compiler_params=pltpu.CompilerParams(
            dimension_semantics=("arbitrary",)),
    )(mp, mn)

    labels_t, deltas_t = pl.pallas_call(
        _assign_kernel,
        grid=(nt,),
        in_specs=[
            pl.BlockSpec((b, 1, 4, tile), lambda ki: (0, ki, 0, 0)),
            pl.BlockSpec((b, 1, 1, tile), lambda ki: (0, ki, 0, 0)),
            pl.BlockSpec((b, 1, 1, tile), lambda ki: (0, ki, 0, 0)),
            pl.BlockSpec((b, 1, 5, tile), lambda ki: (0, ki, 0, 0)),
            pl.BlockSpec((b, 1, 4), lambda ki: (0, 0, 0)),
        ],
        out_specs=[
            pl.BlockSpec((_NUM_LABELS, b, tile), lambda ki: (0, 0, ki)),
            pl.BlockSpec((b, _NUM_LABELS, 4, tile), lambda ki: (0, 0, 0, ki)),
        ],
        out_shape=[
            jax.ShapeDtypeStruct((_NUM_LABELS, b, n), f32),
            jax.ShapeDtypeStruct((b, _NUM_LABELS, 4, n), f32),
        ],
        compiler_params=pltpu.CompilerParams(
            dimension_semantics=("arbitrary",)),
    )(roi_row, mp, mn, pack, thr)

    labels_out = jnp.transpose(labels_t, (1, 2, 0))
    deltas = jnp.transpose(deltas_t, (0, 3, 1, 2))
    return deltas, labels_out
